# baseline (speedup 1.0000x reference)
import functools
import numpy as np

import jax
import jax.numpy as jnp
from jax.experimental import pallas as pl
from jax.experimental.pallas import tpu as pltpu

B, S, D = 1, 2048, 1024
NQ, NKV, HD = 16, 8, 64
E, K, FF = 8, 2, 3584
SW = 512
EPS = 1e-5
HI = jax.lax.Precision.HIGHEST

RB = 512
QB = 512
FFB = 512
TB = 256

def _rope_tables():
    pos = jnp.arange(S, dtype=jnp.float32)
    inv = 1.0 / (10000.0 ** (jnp.arange(0, HD, 2, dtype=jnp.float32) / HD))
    f = pos[:, None] * inv[None, :]
    cos64 = jnp.concatenate([jnp.cos(f), jnp.cos(f)], axis=1)
    sin64 = jnp.concatenate([jnp.sin(f), jnp.sin(f)], axis=1)
    return (jnp.tile(cos64, (1, NQ)), jnp.tile(sin64, (1, NQ)),
            jnp.tile(cos64, (1, NKV)), jnp.tile(sin64, (1, NKV)))


def _dot(a, b, prec=HI):
    return jax.lax.dot_general(a, b, (((1,), (0,)), ((), ())),
                               precision=prec, preferred_element_type=jnp.float32)


def _dot_t(a, b, prec=HI):
    return jax.lax.dot_general(a, b, (((1,), (1,)), ((), ())),
                               precision=prec, preferred_element_type=jnp.float32)


def _rope(t, cos, sin):
    half = HD // 2
    lane = jax.lax.broadcasted_iota(jnp.int32, t.shape, 1) % HD
    rol_m = jnp.concatenate([t[:, half:], t[:, :half]], axis=1)
    rol_p = jnp.concatenate([t[:, -half:], t[:, :-half]], axis=1)
    rot = jnp.where(lane < half, -rol_m, rol_p)
    return t * cos + rot * sin


def _a_body(x_ref, w_ref, nw_ref, cq_ref, sq_ref, ck_ref, sk_ref,
            q_ref, k_ref, v_ref):
    x = x_ref[...]
    var = jnp.mean(x * x, axis=1, keepdims=True)
    h = x * jax.lax.rsqrt(var + EPS) * nw_ref[...]
    qkv = _dot(h, w_ref[...])
    q = qkv[:, :NQ * HD]
    k = qkv[:, NQ * HD:(NQ + NKV) * HD]
    v = qkv[:, (NQ + NKV) * HD:]
    q_ref[...] = _rope(q, cq_ref[...], sq_ref[...])
    k_ref[...] = _rope(k, ck_ref[...], sk_ref[...])
    v_ref[...] = v


def _run_a(x2d, wqkv, nw):
    cosq, sinq, cosk, sink = _rope_tables()
    return pl.pallas_call(
        _a_body,
        grid=(S // RB,),
        in_specs=[
            pl.BlockSpec((RB, D), lambda i: (i, 0)),
            pl.BlockSpec((D, (NQ + 2 * NKV) * HD), lambda i: (0, 0)),
            pl.BlockSpec((1, D), lambda i: (0, 0)),
            pl.BlockSpec((RB, NQ * HD), lambda i: (i, 0)),
            pl.BlockSpec((RB, NQ * HD), lambda i: (i, 0)),
            pl.BlockSpec((RB, NKV * HD), lambda i: (i, 0)),
            pl.BlockSpec((RB, NKV * HD), lambda i: (i, 0)),
        ],
        out_specs=[
            pl.BlockSpec((RB, NQ * HD), lambda i: (i, 0)),
            pl.BlockSpec((RB, NKV * HD), lambda i: (i, 0)),
            pl.BlockSpec((RB, NKV * HD), lambda i: (i, 0)),
        ],
        out_shape=[
            jax.ShapeDtypeStruct((S, NQ * HD), jnp.float32),
            jax.ShapeDtypeStruct((S, NKV * HD), jnp.float32),
            jax.ShapeDtypeStruct((S, NKV * HD), jnp.float32),
        ],
    )(x2d, wqkv, nw, cosq, sinq, cosk, sink)


def _b_body(q_ref, kp_ref, kc_ref, vp_ref, vc_ref, o_ref):
    qb = pl.program_id(1)
    row = qb * QB + jax.lax.broadcasted_iota(jnp.int32, (QB, QB), 0)
    col_c = qb * QB + jax.lax.broadcasted_iota(jnp.int32, (QB, QB), 1)
    col_p = col_c - QB
    mask_c = (col_c <= row) & (col_c > row - SW)
    mask_p = (col_p >= 0) & (col_p <= row) & (col_p > row - SW)
    scale = np.float32(1.0 / np.sqrt(HD))
    for qi in range(2):
        qh = q_ref[qi]
        sp = jnp.where(mask_p, _dot_t(qh, kp_ref[0]) * scale, -1e9)
        sc = jnp.where(mask_c, _dot_t(qh, kc_ref[0]) * scale, -1e9)
        m = jnp.maximum(jnp.max(sp, axis=1, keepdims=True),
                        jnp.max(sc, axis=1, keepdims=True))
        ep = jnp.exp(sp - m)
        ec = jnp.exp(sc - m)
        inv = 1.0 / (jnp.sum(ep, axis=1, keepdims=True)
                     + jnp.sum(ec, axis=1, keepdims=True))
        o = _dot(ep * inv, vp_ref[0]) + _dot(ec * inv, vc_ref[0])
        o_ref[qi] = o


def _run_b(q3, k3, v3):
    return pl.pallas_call(
        _b_body,
        grid=(NKV, S // QB),
        in_specs=[
            pl.BlockSpec((2, QB, HD), lambda h, qb: (h, qb, 0)),
            pl.BlockSpec((1, QB, HD), lambda h, qb: (h, jnp.maximum(qb - 1, 0), 0)),
            pl.BlockSpec((1, QB, HD), lambda h, qb: (h, qb, 0)),
            pl.BlockSpec((1, QB, HD), lambda h, qb: (h, jnp.maximum(qb - 1, 0), 0)),
            pl.BlockSpec((1, QB, HD), lambda h, qb: (h, qb, 0)),
        ],
        out_specs=pl.BlockSpec((2, QB, HD), lambda h, qb: (h, qb, 0)),
        out_shape=jax.ShapeDtypeStruct((NQ, S, HD), jnp.float32),
    )(q3, k3, k3, v3, v3)


def _c_body(attn_ref, ow_ref, x_ref, fw_ref, rw_ref,
            x2_ref, h2_ref, lg_ref, i0_ref, i1_ref, w1_ref, w2_ref):
    x2 = x_ref[...] + _dot(attn_ref[...], ow_ref[...])
    var = jnp.mean(x2 * x2, axis=1, keepdims=True)
    h2 = x2 * jax.lax.rsqrt(var + EPS) * fw_ref[...]
    lg = _dot(h2, rw_ref[...])
    lanes = jax.lax.broadcasted_iota(jnp.int32, lg.shape, 1)
    neg = jnp.float32(-jnp.inf)
    lgm = jnp.where(lanes < E, lg, neg)
    m1 = jnp.max(lgm, axis=1, keepdims=True)
    i1 = jnp.min(jnp.where(lgm == m1, lanes, 999), axis=1, keepdims=True)
    masked = jnp.where(lanes == i1, neg, lgm)
    m2 = jnp.max(masked, axis=1, keepdims=True)
    i2 = jnp.min(jnp.where(masked == m2, lanes, 999), axis=1, keepdims=True)
    e2 = jnp.exp(m2 - m1)
    den = 1.0 + e2
    x2_ref[...] = x2
    h2_ref[...] = h2
    lg_ref[...] = lg
    i0_ref[...] = jnp.broadcast_to(i1, lg.shape)
    i1_ref[...] = jnp.broadcast_to(i2, lg.shape)
    w1_ref[...] = jnp.broadcast_to(1.0 / den, lg.shape)
    w2_ref[...] = jnp.broadcast_to(e2 / den, lg.shape)


def _run_c(attn, o_w, x2d, fw, rwp):
    outs = [
        jax.ShapeDtypeStruct((S, D), jnp.float32),
        jax.ShapeDtypeStruct((S, D), jnp.float32),
        jax.ShapeDtypeStruct((S, 128), jnp.float32),
        jax.ShapeDtypeStruct((S, 128), jnp.int32),
        jax.ShapeDtypeStruct((S, 128), jnp.int32),
        jax.ShapeDtypeStruct((S, 128), jnp.float32),
        jax.ShapeDtypeStruct((S, 128), jnp.float32),
    ]
    return pl.pallas_call(
        _c_body,
        grid=(S // RB,),
        in_specs=[
            pl.BlockSpec((RB, NQ * HD), lambda i: (i, 0)),
            pl.BlockSpec((NQ * HD, D), lambda i: (0, 0)),
            pl.BlockSpec((RB, D), lambda i: (i, 0)),
            pl.BlockSpec((1, D), lambda i: (0, 0)),
            pl.BlockSpec((D, 128), lambda i: (0, 0)),
        ],
        out_specs=[
            pl.BlockSpec((RB, D), lambda i: (i, 0)),
            pl.BlockSpec((RB, D), lambda i: (i, 0)),
            pl.BlockSpec((RB, 128), lambda i: (i, 0)),
            pl.BlockSpec((RB, 128), lambda i: (i, 0)),
            pl.BlockSpec((RB, 128), lambda i: (i, 0)),
            pl.BlockSpec((RB, 128), lambda i: (i, 0)),
            pl.BlockSpec((RB, 128), lambda i: (i, 0)),
        ],
        out_shape=outs,
    )(attn, o_w, x2d, fw, rwp)


def _d_body(h2_ref, x2_ref, i0_ref, i1_ref, w1_ref, w2_ref,
            wg_ref, wi_ref, wo_ref, out_ref):
    e = pl.program_id(0)
    f = pl.program_id(1)
    wg = wg_ref[0].astype(jnp.bfloat16)
    wi = wi_ref[0].astype(jnp.bfloat16)
    wo = wo_ref[0].astype(jnp.bfloat16)
    first = (e == 0) & (f == 0)
    for t in range(S // TB):
        sl = pl.ds(t * TB, TB)
        h2 = h2_ref[sl, :].astype(jnp.bfloat16)
        gate = _dot(h2, wg, prec=None)
        gate = gate * jax.nn.sigmoid(gate)
        inter = _dot(h2, wi, prec=None)
        prod = (inter * gate).astype(jnp.bfloat16)
        eo = _dot(prod, wo, prec=None)
        c = (jnp.where(i0_ref[sl, :1] == e, w1_ref[sl, :1], 0.0)
             + jnp.where(i1_ref[sl, :1] == e, w2_ref[sl, :1], 0.0))
        contrib = eo * c

        @pl.when(first)
        def _():
            out_ref[sl, :] = x2_ref[sl, :] + contrib

        @pl.when(jnp.logical_not(first))
        def _():
            out_ref[sl, :] += contrib


def _run_d(h2, x2, i0, i1, w1, w2, w_gate, w_inter, w_out):
    return pl.pallas_call(
        _d_body,
        grid=(E, FF // FFB),
        in_specs=[
            pl.BlockSpec((S, D), lambda e, f: (0, 0)),
            pl.BlockSpec((S, D), lambda e, f: (0, 0)),
            pl.BlockSpec((S, 128), lambda e, f: (0, 0)),
            pl.BlockSpec((S, 128), lambda e, f: (0, 0)),
            pl.BlockSpec((S, 128), lambda e, f: (0, 0)),
            pl.BlockSpec((S, 128), lambda e, f: (0, 0)),
            pl.BlockSpec((1, D, FFB), lambda e, f: (e, 0, f)),
            pl.BlockSpec((1, D, FFB), lambda e, f: (e, 0, f)),
            pl.BlockSpec((1, FFB, D), lambda e, f: (e, f, 0)),
        ],
        out_specs=pl.BlockSpec((S, D), lambda e, f: (0, 0)),
        out_shape=jax.ShapeDtypeStruct((S, D), jnp.float32),
    )(h2, x2, i0, i1, w1, w2, w_gate, w_inter, w_out)


def kernel(decoder_sequence, attn_norm_w, q_w, k_w, v_w, o_w, ffn_norm_w,
           router_w, w_gate, w_inter, w_out):
    x2d = decoder_sequence.reshape(S, D)
    wqkv = jnp.concatenate([q_w, k_w, v_w], axis=1)
    nw = attn_norm_w.reshape(1, D)
    fw = ffn_norm_w.reshape(1, D)
    rwp = jnp.pad(router_w, ((0, 0), (0, 128 - E)))

    q, k, v = _run_a(x2d, wqkv, nw)
    q3 = q.reshape(S, NQ, HD).transpose(1, 0, 2)
    k3 = k.reshape(S, NKV, HD).transpose(1, 0, 2)
    v3 = v.reshape(S, NKV, HD).transpose(1, 0, 2)
    attn3 = _run_b(q3, k3, v3)
    attn = attn3.transpose(1, 0, 2).reshape(S, NQ * HD)
    x2, h2, lg, i0, i1, w1, w2 = _run_c(attn, o_w, x2d, fw, rwp)
    out2d = _run_d(h2, x2, i0, i1, w1, w2, w_gate, w_inter, w_out)
    return out2d.reshape(B, S, D), lg[:, :E]
